# baseline (speedup 1.0000x reference)
import functools

import jax
import jax.numpy as jnp
from jax import lax
from jax.experimental import pallas as pl
from jax.experimental.pallas import tpu as pltpu
from jax.experimental.pallas import tpu_sc as plsc


def kernel(x, row_embed):
    n = x.shape[1]
    d = row_embed.shape[1]

    info = plsc.get_sparse_core_info()
    nc, ns = info.num_cores, info.num_subcores
    nw = nc * ns
    rows_per_w = n // nw
    rows_c = min(32, rows_per_w)
    nchunks = rows_per_w // rows_c
    nbuf = max(1, nchunks - 1)

    mesh = plsc.VectorSubcoreMesh(core_axis_name="c", subcore_axis_name="s")

    @functools.partial(
        pl.kernel,
        mesh=mesh,
        out_type=jax.ShapeDtypeStruct((n, d), row_embed.dtype),
        scratch_types=(
            [pltpu.VMEM((nbuf, rows_c, d), jnp.float32)]
            + [pltpu.SemaphoreType.DMA] * (2 * nbuf)
        ),
    )
    def copy_rows(table_hbm, out_hbm, buf, *sems):
        wid = lax.axis_index("s") * nc + lax.axis_index("c")
        base = wid * rows_per_w
        in_sems = sems[:nbuf]
        out_sems = sems[nbuf:]

        def chunk(r, i):
            return r.at[pl.ds(base + i * rows_c, rows_c)]

        in_copies = [None] * nbuf
        out_copies = [None] * nbuf
        for b in range(min(nbuf, nchunks)):
            in_copies[b] = pltpu.async_copy(
                chunk(table_hbm, b), buf.at[b], in_sems[b])
        for i in range(nchunks):
            b = i % nbuf
            in_copies[b].wait()
            out_copies[b] = pltpu.async_copy(
                buf.at[b], chunk(out_hbm, i), out_sems[b])
            nxt = i + nbuf
            if nxt < nchunks:
                out_copies[b].wait()
                out_copies[b] = None
                in_copies[b] = pltpu.async_copy(
                    chunk(table_hbm, nxt), buf.at[b], in_sems[b])
        for b in range(nbuf):
            if out_copies[b] is not None:
                out_copies[b].wait()

    return copy_rows(row_embed)[None]


# speedup vs baseline: 1.0135x; 1.0135x over previous
import functools

import jax
import jax.numpy as jnp
from jax import lax
from jax.experimental import pallas as pl
from jax.experimental.pallas import tpu as pltpu
from jax.experimental.pallas import tpu_sc as plsc


def kernel(x, row_embed):
    n = x.shape[1]
    d = row_embed.shape[1]

    info = plsc.get_sparse_core_info()
    nc, ns = info.num_cores, info.num_subcores
    nw = nc * ns
    rows_per_w = n // nw
    rows_c = min(8, rows_per_w)
    nchunks = rows_per_w // rows_c
    nbuf = max(1, nchunks - 1)

    mesh = plsc.VectorSubcoreMesh(core_axis_name="c", subcore_axis_name="s")

    @functools.partial(
        pl.kernel,
        mesh=mesh,
        out_type=jax.ShapeDtypeStruct((n, d), row_embed.dtype),
        scratch_types=(
            [pltpu.VMEM((nbuf, rows_c, d), jnp.float32)]
            + [pltpu.SemaphoreType.DMA] * (2 * nbuf)
        ),
    )
    def copy_rows(table_hbm, out_hbm, buf, *sems):
        wid = lax.axis_index("s") * nc + lax.axis_index("c")
        base = wid * rows_per_w
        in_sems = sems[:nbuf]
        out_sems = sems[nbuf:]

        def chunk(r, i):
            return r.at[pl.ds(base + i * rows_c, rows_c)]

        in_copies = [None] * nbuf
        out_copies = [None] * nbuf
        for b in range(min(nbuf, nchunks)):
            in_copies[b] = pltpu.async_copy(
                chunk(table_hbm, b), buf.at[b], in_sems[b])
        for i in range(nchunks):
            b = i % nbuf
            in_copies[b].wait()
            out_copies[b] = pltpu.async_copy(
                buf.at[b], chunk(out_hbm, i), out_sems[b])
            nxt = i + nbuf
            if nxt < nchunks:
                out_copies[b].wait()
                out_copies[b] = None
                in_copies[b] = pltpu.async_copy(
                    chunk(table_hbm, nxt), buf.at[b], in_sems[b])
        for b in range(nbuf):
            if out_copies[b] is not None:
                out_copies[b].wait()

    return copy_rows(row_embed)[None]


# speedup vs baseline: 1.0332x; 1.0194x over previous
import functools

import jax
import jax.numpy as jnp
from jax import lax
from jax.experimental import pallas as pl
from jax.experimental.pallas import tpu as pltpu
from jax.experimental.pallas import tpu_sc as plsc


def kernel(x, row_embed):
    n = x.shape[1]
    d = row_embed.shape[1]

    info = plsc.get_sparse_core_info()
    nc, ns = info.num_cores, info.num_subcores
    nw = nc * ns
    rows_per_w = n // nw
    rows_c = min(16, rows_per_w)
    nchunks = rows_per_w // rows_c
    nbuf = max(1, nchunks - 1)

    mesh = plsc.VectorSubcoreMesh(core_axis_name="c", subcore_axis_name="s")

    @functools.partial(
        pl.kernel,
        mesh=mesh,
        out_type=jax.ShapeDtypeStruct((n, d), row_embed.dtype),
        scratch_types=(
            [pltpu.VMEM((nbuf, rows_c, d), jnp.float32)]
            + [pltpu.SemaphoreType.DMA] * (2 * nbuf)
        ),
    )
    def copy_rows(table_hbm, out_hbm, buf, *sems):
        wid = lax.axis_index("s") * nc + lax.axis_index("c")
        base = wid * rows_per_w
        in_sems = sems[:nbuf]
        out_sems = sems[nbuf:]

        def chunk(r, i):
            return r.at[pl.ds(base + i * rows_c, rows_c)]

        in_copies = [None] * nbuf
        out_copies = [None] * nbuf
        for b in range(min(nbuf, nchunks)):
            in_copies[b] = pltpu.async_copy(
                chunk(table_hbm, b), buf.at[b], in_sems[b])
        for i in range(nchunks):
            b = i % nbuf
            in_copies[b].wait()
            out_copies[b] = pltpu.async_copy(
                buf.at[b], chunk(out_hbm, i), out_sems[b])
            nxt = i + nbuf
            if nxt < nchunks:
                out_copies[b].wait()
                out_copies[b] = None
                in_copies[b] = pltpu.async_copy(
                    chunk(table_hbm, nxt), buf.at[b], in_sems[b])
        for b in range(nbuf):
            if out_copies[b] is not None:
                out_copies[b].wait()

    return copy_rows(row_embed)[None]
